# baseline (speedup 1.0000x reference)
import jax
import jax.numpy as jnp
from jax import lax
from jax.experimental import pallas as pl
from jax.experimental.pallas import tpu as pltpu
from jax.experimental.pallas import tpu_sc as plsc

SEQ = 64
EMBED = 64
NJ = 21
NJP = 32
NT = SEQ * NJP
TW = 128
NC = 2
NS = 16
NW = NC * NS
NBUF = 4


def _sc_body(p_hbm, c_hbm, pos_hbm, piece_hbm, color_hbm, out_hbm, tbl_hbm,
             pos_v, piece_v, color_v, joint_v, loc_v,
             p_v, c_v, idx_v, gbuf, wbuf, gsem, wsem):
    cid = lax.axis_index("c")
    sid = lax.axis_index("s")
    wid = sid * NC + cid
    brows_per_tile = 4096 // NW

    pltpu.sync_copy(pos_hbm, pos_v)
    pltpu.sync_copy(piece_hbm, piece_v)
    pltpu.sync_copy(color_hbm, color_v)

    def build_joint(j, _):
        pj = jnp.minimum(j // 3, 6)
        cj = j - (j // 3) * 3
        for k in range(EMBED // 16):
            joint_v[j, pl.ds(k * 16, 16)] = (
                piece_v[pj, pl.ds(k * 16, 16)] + color_v[cj, pl.ds(k * 16, 16)])
        return _
    lax.fori_loop(0, NJP, build_joint, 0)

    def build_row(i, _):
        r = sid * (NT // NS) + i
        s = r // NJP
        j = r - s * NJP
        for k in range(EMBED // 16):
            v = pos_v[s, pl.ds(k * 16, 16)] + joint_v[j, pl.ds(k * 16, 16)]
            loc_v[i, pl.ds(k * 16, 16)] = v
            loc_v[i, pl.ds(EMBED + k * 16, 16)] = v
        return _
    lax.fori_loop(0, NT // NS, build_row, 0)
    pltpu.sync_copy(loc_v, tbl_hbm.at[pl.ds(cid * NT + sid * (NT // NS), NT // NS)])

    base_b = wid * brows_per_tile
    pltpu.sync_copy(p_hbm.at[pl.ds(base_b, brows_per_tile)], p_v)
    pltpu.sync_copy(c_hbm.at[pl.ds(base_b, brows_per_tile)], c_v)
    tbl_base = cid * NT

    def build_idx(i, _):
        r = i // 4
        k = i - r * 4
        pv = p_v[r, pl.ds(k * 16, 16)]
        cv = c_v[r, pl.ds(k * 16, 16)]
        sv = lax.broadcasted_iota(jnp.int32, (16,), 0) + k * 16
        idx_v[r, pl.ds(k * 16, 16)] = sv * NJP + pv * 3 + cv + tbl_base
        return _
    lax.fori_loop(0, brows_per_tile * 4, build_idx, 0)

    plsc.subcore_barrier()

    def gather_cp(r, b):
        return pltpu.make_async_copy(
            tbl_hbm.at[idx_v.at[r]], gbuf.at[b], gsem.at[b])

    def write_cp(r, w):
        return pltpu.make_async_copy(
            wbuf.at[w], out_hbm.at[base_b + r], wsem.at[w])

    for b in range(NBUF - 1):
        gather_cp(b, b).start()

    def chunk_step(r, _):
        b = lax.rem(r, NBUF)
        w = r & 1
        gather_cp(r, b).wait()

        @pl.when(r + NBUF - 1 < brows_per_tile)
        def _start_ahead():
            gather_cp(r + NBUF - 1, lax.rem(r + NBUF - 1, NBUF)).start()

        @pl.when(r >= 2)
        def _drain_write():
            write_cp(r - 2, w).wait()

        def repack(i, _):
            for k in range(EMBED // 16):
                wbuf[w, i, pl.ds(k * 16, 16)] = gbuf[b, i, pl.ds(k * 16, 16)]
            return _
        lax.fori_loop(0, SEQ, repack, 0)

        write_cp(r, w).start()
        return _
    lax.fori_loop(0, brows_per_tile, chunk_step, 0)
    write_cp(brows_per_tile - 2, 0).wait()
    write_cp(brows_per_tile - 1, 1).wait()


def kernel(pieces_ids, color_ids, position_emb, piece_emb, color_emb):
    B = pieces_ids.shape[0]
    p32 = pieces_ids.astype(jnp.int32)
    c32 = color_ids.astype(jnp.int32)

    mesh = plsc.VectorSubcoreMesh(core_axis_name="c", subcore_axis_name="s")
    run = pl.kernel(
        _sc_body,
        mesh=mesh,
        out_type=(
            jax.ShapeDtypeStruct((B, SEQ, EMBED), jnp.float32),
            jax.ShapeDtypeStruct((NC * NT, TW), jnp.float32),
        ),
        scratch_types=[
            pltpu.VMEM((SEQ, EMBED), jnp.float32),
            pltpu.VMEM((7, EMBED), jnp.float32),
            pltpu.VMEM((3, EMBED), jnp.float32),
            pltpu.VMEM((NJP, EMBED), jnp.float32),
            pltpu.VMEM((NT // NS, TW), jnp.float32),
            pltpu.VMEM((B // NW, SEQ), jnp.int32),
            pltpu.VMEM((B // NW, SEQ), jnp.int32),
            pltpu.VMEM((B // NW, SEQ), jnp.int32),
            pltpu.VMEM((NBUF, SEQ, TW), jnp.float32),
            pltpu.VMEM((2, SEQ, EMBED), jnp.float32),
            pltpu.SemaphoreType.DMA((NBUF,)),
            pltpu.SemaphoreType.DMA((2,)),
        ],
    )
    out, _tbl = run(p32, c32, position_emb, piece_emb, color_emb)
    return out


# speedup vs baseline: 9.7739x; 9.7739x over previous
import jax
import jax.numpy as jnp
from jax import lax
from jax.experimental import pallas as pl
from jax.experimental.pallas import tpu as pltpu

SEQ = 64
EMBED = 64
NJ = 21
KJ = 32
G = 8
NG = SEQ // G
BB = 512


def _body(pT_ref, cT_ref, pos_ref, pieceT_ref, colorT_ref, out_ref, jt_ref):
    @pl.when(pl.program_id(0) == 0)
    def _build_tables():
        selp = (lax.broadcasted_iota(jnp.int32, (7, KJ), 0)
                == lax.broadcasted_iota(jnp.int32, (7, KJ), 1) // 3)
        selc = ((lax.broadcasted_iota(jnp.int32, (3, KJ), 0)
                 == lax.broadcasted_iota(jnp.int32, (3, KJ), 1) % 3)
                & (lax.broadcasted_iota(jnp.int32, (3, KJ), 1) < NJ))
        joint_t = (
            jnp.dot(pieceT_ref[...], selp.astype(jnp.float32),
                    preferred_element_type=jnp.float32)
            + jnp.dot(colorT_ref[...], selc.astype(jnp.float32),
                      preferred_element_type=jnp.float32))
        jtr = jnp.broadcast_to(joint_t[None], (G, EMBED, KJ)).reshape(
            G * EMBED, KJ)
        rexp = (lax.broadcasted_iota(jnp.int32, (KJ, G * KJ), 0)
                == lax.broadcasted_iota(jnp.int32, (KJ, G * KJ), 1) // G)
        a = jnp.dot(jtr, rexp.astype(jnp.float32),
                    preferred_element_type=jnp.float32)
        mask = (lax.broadcasted_iota(jnp.int32, (G * EMBED, G * KJ), 1) % G
                == lax.broadcasted_iota(jnp.int32, (G * EMBED, G * KJ), 0)
                // EMBED)
        esel = (lax.broadcasted_iota(jnp.int32, (G * EMBED, G), 0) // EMBED
                == lax.broadcasted_iota(jnp.int32, (G * EMBED, G), 1)
                ).astype(jnp.float32)
        dmask = (lax.broadcasted_iota(jnp.int32, (G * EMBED, EMBED), 1)
                 == lax.broadcasted_iota(jnp.int32, (G * EMBED, EMBED), 0)
                 % EMBED)
        for g in range(NG):
            p2 = jnp.dot(esel, pos_ref[g * G:(g + 1) * G, :],
                         preferred_element_type=jnp.float32)
            posg = jnp.sum(jnp.where(dmask, p2, 0.0), axis=1, keepdims=True)
            jt_ref[g] = jnp.where(mask, a + posg, 0.0)

    j2 = pT_ref[...] * 3 + cT_ref[...]
    jsel = lax.broadcasted_iota(jnp.int32, (G * KJ, BB), 0) // G
    for g in range(NG):
        j2g = j2[g * G:(g + 1) * G, :]
        oh = (jnp.broadcast_to(j2g[None], (KJ, G, BB)).reshape(G * KJ, BB)
              == jsel).astype(jnp.float32)
        m = jnp.dot(jt_ref[g], oh, preferred_element_type=jnp.float32)
        out_ref[g * G:(g + 1) * G] = m.reshape(G, EMBED, BB)


def kernel(pieces_ids, color_ids, position_emb, piece_emb, color_emb):
    B = pieces_ids.shape[0]
    pT = pieces_ids.astype(jnp.int32).T
    cT = color_ids.astype(jnp.int32).T
    out_t = pl.pallas_call(
        _body,
        grid=(B // BB,),
        in_specs=[
            pl.BlockSpec((SEQ, BB), lambda i: (0, i)),
            pl.BlockSpec((SEQ, BB), lambda i: (0, i)),
            pl.BlockSpec((SEQ, EMBED), lambda i: (0, 0)),
            pl.BlockSpec((EMBED, 7), lambda i: (0, 0)),
            pl.BlockSpec((EMBED, 3), lambda i: (0, 0)),
        ],
        out_specs=pl.BlockSpec((SEQ, EMBED, BB), lambda i: (0, 0, i)),
        out_shape=jax.ShapeDtypeStruct((SEQ, EMBED, B), jnp.float32),
        scratch_shapes=[
            pltpu.VMEM((NG, G * EMBED, G * KJ), jnp.float32),
        ],
    )(pT, cT, position_emb, piece_emb.T, color_emb.T)
    return jnp.transpose(out_t, (2, 0, 1))
